# Initial kernel scaffold; baseline (speedup 1.0000x reference)
import jax
import jax.numpy as jnp
from jax.experimental import pallas as pl


def kernel(x, edge_attr, e1_W1, e1_b1, e1_W2, e1_b2, c1_Wl, c1_bl, c1_Wr, c1_br, c1_We, c1_att, c1_bias, bn_n_g, bn_n_b, bn_n_rm, bn_n_rv, bn_e_g, bn_e_b, bn_e_rm, bn_e_rv, e2_W1, e2_b1, e2_W2, e2_b2, c2_Wl, c2_bl, c2_Wr, c2_br, c2_We, c2_att, c2_bias, fc1_W, fc1_b, fc2_W, fc2_b, edge_index, batch):
    raise NotImplementedError("write your pallas kernel here")


# speedup vs baseline: 6.7943x; 6.7943x over previous
import functools
import jax
import jax.numpy as jnp
from jax.experimental import pallas as pl
from jax.experimental.pallas import tpu as pltpu

F32 = jnp.float32


def _mm(a, b):
    return jnp.dot(a, b, preferred_element_type=F32)


def _leaky(v):
    return jnp.where(v > 0, v, 0.2 * v)


def _nodeproj_body(x_ref, w_ref, b_ref, o_ref):
    o_ref[...] = _mm(x_ref[...], w_ref[...]) + b_ref[...]


def _node_proj(x, w, b, bn):
    n, k = x.shape
    m = w.shape[1]
    grid = (n // bn,)
    return pl.pallas_call(
        _nodeproj_body,
        grid=grid,
        in_specs=[
            pl.BlockSpec((bn, k), lambda i: (i, 0)),
            pl.BlockSpec((k, m), lambda i: (0, 0)),
            pl.BlockSpec((1, m), lambda i: (0, 0)),
        ],
        out_specs=pl.BlockSpec((bn, m), lambda i: (i, 0)),
        out_shape=jax.ShapeDtypeStruct((n, m), F32),
    )(x, w, b.reshape(1, m))


def _edgeA1_body(gs_ref, gd_ref, ea_ref, w1c_ref, b1_ref, w2_ref, b2_ref,
                 we_ref, atts_ref, ea1_ref, al_ref, *, hc):
    g = jnp.maximum(gs_ref[:, hc:] + gd_ref[:, hc:]
                    + _mm(ea_ref[...], w1c_ref[...]) + b1_ref[...], 0.0)
    ea1 = _mm(g, w2_ref[...]) + b2_ref[...]
    ea1_ref[...] = ea1
    m = _leaky(gs_ref[:, :hc] + gd_ref[:, :hc] + _mm(ea1, we_ref[...]))
    al_ref[...] = _mm(m, atts_ref[...])


def _edge_a1(gs, gd, eattr, w1c, b1, w2, b2, we, atts, be, hc):
    e = gs.shape[0]
    wgs = gs.shape[1]
    de = eattr.shape[1]
    d1 = w2.shape[0]
    d2 = w2.shape[1]
    grid = (e // be,)
    return pl.pallas_call(
        functools.partial(_edgeA1_body, hc=hc),
        grid=grid,
        in_specs=[
            pl.BlockSpec((be, wgs), lambda i: (i, 0)),
            pl.BlockSpec((be, wgs), lambda i: (i, 0)),
            pl.BlockSpec((be, de), lambda i: (i, 0)),
            pl.BlockSpec((de, d1), lambda i: (0, 0)),
            pl.BlockSpec((1, d1), lambda i: (0, 0)),
            pl.BlockSpec((d1, d2), lambda i: (0, 0)),
            pl.BlockSpec((1, d2), lambda i: (0, 0)),
            pl.BlockSpec((d2, hc), lambda i: (0, 0)),
            pl.BlockSpec((hc, 8), lambda i: (0, 0)),
        ],
        out_specs=[
            pl.BlockSpec((be, d2), lambda i: (i, 0)),
            pl.BlockSpec((be, 8), lambda i: (i, 0)),
        ],
        out_shape=[
            jax.ShapeDtypeStruct((e, d2), F32),
            jax.ShapeDtypeStruct((e, 8), F32),
        ],
    )(gs, gd, eattr, w1c, b1.reshape(1, d1), w2, b2.reshape(1, d2), we, atts)


def _edgeA2_body(gs_ref, gd_ref, ea1_ref, bng_ref, bnb_ref, bnm_ref, bnv_ref,
                 w1c_ref, b1_ref, w2_ref, b2_ref, we_ref, atts_ref, al_ref,
                 *, hc):
    eb = ((ea1_ref[...] - bnm_ref[...])
          * jax.lax.rsqrt(bnv_ref[...] + 1e-5) * bng_ref[...] + bnb_ref[...])
    g = jnp.maximum(gs_ref[:, hc:] + gd_ref[:, hc:]
                    + _mm(eb, w1c_ref[...]) + b1_ref[...], 0.0)
    ea2 = _mm(g, w2_ref[...]) + b2_ref[...]
    m = _leaky(gs_ref[:, :hc] + gd_ref[:, :hc] + _mm(ea2, we_ref[...]))
    al_ref[...] = _mm(m, atts_ref[...])


def _edge_a2(gs, gd, ea1, bng, bnb, bnm, bnv, w1c, b1, w2, b2, we, atts,
             be, hc):
    e = gs.shape[0]
    wgs = gs.shape[1]
    d0 = ea1.shape[1]
    d1 = w2.shape[0]
    d2 = w2.shape[1]
    grid = (e // be,)
    row = lambda a: a.reshape(1, -1)
    return pl.pallas_call(
        functools.partial(_edgeA2_body, hc=hc),
        grid=grid,
        in_specs=[
            pl.BlockSpec((be, wgs), lambda i: (i, 0)),
            pl.BlockSpec((be, wgs), lambda i: (i, 0)),
            pl.BlockSpec((be, d0), lambda i: (i, 0)),
            pl.BlockSpec((1, d0), lambda i: (0, 0)),
            pl.BlockSpec((1, d0), lambda i: (0, 0)),
            pl.BlockSpec((1, d0), lambda i: (0, 0)),
            pl.BlockSpec((1, d0), lambda i: (0, 0)),
            pl.BlockSpec((d0, d1), lambda i: (0, 0)),
            pl.BlockSpec((1, d1), lambda i: (0, 0)),
            pl.BlockSpec((d1, d2), lambda i: (0, 0)),
            pl.BlockSpec((1, d2), lambda i: (0, 0)),
            pl.BlockSpec((d2, hc), lambda i: (0, 0)),
            pl.BlockSpec((hc, 8), lambda i: (0, 0)),
        ],
        out_specs=pl.BlockSpec((be, 8), lambda i: (i, 0)),
        out_shape=jax.ShapeDtypeStruct((e, 8), F32),
    )(gs, gd, ea1, row(bng), row(bnb), row(bnm), row(bnv),
      w1c, row(b1), w2, row(b2), we, atts)


def _edgeB_body(gs_ref, al_ref, sh_ref, es_ref, msk_ref, out_ref, *, hc):
    w8 = jnp.exp(al_ref[...] - sh_ref[...]) * msk_ref[...]
    wexp = _mm(w8, es_ref[...])
    out_ref[:, :hc] = gs_ref[:, :hc] * wexp
    out_ref[:, hc:] = w8


def _edge_b(gs, al8, sh8, es8, msk8, be, hc):
    e = gs.shape[0]
    wgs = gs.shape[1]
    grid = (e // be,)
    return pl.pallas_call(
        functools.partial(_edgeB_body, hc=hc),
        grid=grid,
        in_specs=[
            pl.BlockSpec((be, wgs), lambda i: (i, 0)),
            pl.BlockSpec((be, 8), lambda i: (i, 0)),
            pl.BlockSpec((be, 8), lambda i: (i, 0)),
            pl.BlockSpec((8, hc), lambda i: (0, 0)),
            pl.BlockSpec((1, 8), lambda i: (0, 0)),
        ],
        out_specs=pl.BlockSpec((be, hc + 8), lambda i: (i, 0)),
        out_shape=jax.ShapeDtypeStruct((e, hc + 8), F32),
    )(gs, al8, sh8, es8, msk8)


def _node2_body(s_ref, es_ref, bias_ref, g_ref, b_ref, m_ref, v_ref,
                w_ref, bc_ref, t2_ref, *, hc):
    den = _mm(s_ref[:, hc:], es_ref[...])
    h = jnp.maximum(s_ref[:, :hc] / (den + 1e-16) + bias_ref[...], 0.0)
    h = (h - m_ref[...]) * jax.lax.rsqrt(v_ref[...] + 1e-5) * g_ref[...] \
        + b_ref[...]
    t2_ref[...] = _mm(h, w_ref[...]) + bc_ref[...]


def _node2(s, es8, bias, g, b, m, v, wcat, bcat, bn, hc):
    n = s.shape[0]
    w2 = wcat.shape[1]
    grid = (n // bn,)
    row = lambda a: a.reshape(1, -1)
    return pl.pallas_call(
        functools.partial(_node2_body, hc=hc),
        grid=grid,
        in_specs=[
            pl.BlockSpec((bn, hc + 8), lambda i: (i, 0)),
            pl.BlockSpec((8, hc), lambda i: (0, 0)),
            pl.BlockSpec((1, hc), lambda i: (0, 0)),
            pl.BlockSpec((1, hc), lambda i: (0, 0)),
            pl.BlockSpec((1, hc), lambda i: (0, 0)),
            pl.BlockSpec((1, hc), lambda i: (0, 0)),
            pl.BlockSpec((1, hc), lambda i: (0, 0)),
            pl.BlockSpec((hc, w2), lambda i: (0, 0)),
            pl.BlockSpec((1, w2), lambda i: (0, 0)),
        ],
        out_specs=pl.BlockSpec((bn, w2), lambda i: (i, 0)),
        out_shape=jax.ShapeDtypeStruct((n, w2), F32),
    )(s, es8, row(bias), row(g), row(b), row(m), row(v), wcat, row(bcat))


def _pool_body(s_ref, bt_ref, es_ref, bias_ref, w1_ref, b1_ref, w2_ref,
               b2_ref, o_ref, acc, *, hc, ng, nblk):
    i = pl.program_id(0)

    @pl.when(i == 0)
    def _():
        acc[...] = jnp.zeros_like(acc)

    den = _mm(s_ref[:, hc:], es_ref[...])
    h = jnp.maximum(s_ref[:, :hc] / (den + 1e-16) + bias_ref[...], 0.0)
    seg = bt_ref[0]
    ids = jax.lax.broadcasted_iota(jnp.int32, (ng, seg.shape[1]), 0)
    onehot = (ids == seg).astype(F32)
    acc[...] += _mm(onehot, h)

    @pl.when(i == nblk - 1)
    def _():
        f = jnp.maximum(_mm(acc[...], w1_ref[...]) + b1_ref[...], 0.0)
        o_ref[...] = _mm(f, w2_ref[...]) + b2_ref[...]


def _pool(s, batch3, es8, bias, w1, b1, w2p, b2p, bn, hc, ng):
    n = s.shape[0]
    nblk = n // bn
    dh = w1.shape[1]
    row = lambda a: a.reshape(1, -1)
    return pl.pallas_call(
        functools.partial(_pool_body, hc=hc, ng=ng, nblk=nblk),
        grid=(nblk,),
        in_specs=[
            pl.BlockSpec((bn, hc + 8), lambda i: (i, 0)),
            pl.BlockSpec((1, 1, bn), lambda i: (i, 0, 0)),
            pl.BlockSpec((8, hc), lambda i: (0, 0)),
            pl.BlockSpec((1, hc), lambda i: (0, 0)),
            pl.BlockSpec((hc, dh), lambda i: (0, 0)),
            pl.BlockSpec((1, dh), lambda i: (0, 0)),
            pl.BlockSpec((dh, 8), lambda i: (0, 0)),
            pl.BlockSpec((1, 8), lambda i: (0, 0)),
        ],
        out_specs=pl.BlockSpec((ng, 8), lambda i: (0, 0)),
        out_shape=jax.ShapeDtypeStruct((ng, 8), F32),
        scratch_shapes=[pltpu.VMEM((ng, hc), F32)],
    )(s, batch3, es8, row(bias), w1, row(b1), w2p, row(b2p))


def kernel(x, edge_attr, e1_W1, e1_b1, e1_W2, e1_b2, c1_Wl, c1_bl, c1_Wr,
           c1_br, c1_We, c1_att, c1_bias, bn_n_g, bn_n_b, bn_n_rm, bn_n_rv,
           bn_e_g, bn_e_b, bn_e_rm, bn_e_rv, e2_W1, e2_b1, e2_W2, e2_b2,
           c2_Wl, c2_bl, c2_Wr, c2_br, c2_We, c2_att, c2_bias, fc1_W, fc1_b,
           fc2_W, fc2_b, edge_index, batch):
    n, df = x.shape
    e, de = edge_attr.shape
    h_, c_ = c1_att.shape
    hc = h_ * c_
    ng = 64
    be = 2000
    bn = 1000

    src = edge_index[0]
    dst = edge_index[1]

    eye_h = jnp.kron(jnp.eye(h_, dtype=F32), jnp.ones((1, c_), F32))
    es8 = jnp.concatenate([eye_h, jnp.zeros((8 - h_, hc), F32)], 0)
    msk8 = jnp.concatenate([jnp.ones((h_,), F32),
                            jnp.zeros((8 - h_,), F32)]).reshape(1, 8)

    atts1 = _build_atts(c1_att, h_, c_)
    atts2 = _build_atts(c2_att, h_, c_)

    wcat1 = jnp.concatenate([c1_Wl, e1_W1[:df], c1_Wr, e1_W1[df:2 * df]], 1)
    bcat1 = jnp.concatenate([c1_bl, jnp.zeros((e1_W1.shape[1],), F32),
                             c1_br, jnp.zeros((e1_W1.shape[1],), F32)])
    t1 = _node_proj(x, wcat1, bcat1, bn)
    w1 = hc + e1_W1.shape[1]
    gs1 = jnp.take(t1[:, :w1], src, axis=0)
    gd1 = jnp.take(t1[:, w1:], dst, axis=0)

    ea1, al1 = _edge_a1(gs1, gd1, edge_attr, e1_W1[2 * df:], e1_b1,
                        e1_W2, e1_b2, c1_We, atts1, be, hc)

    amax1 = jax.ops.segment_max(al1[:, :h_], dst, num_segments=n)
    amax1 = jnp.where(jnp.isfinite(amax1), amax1, 0.0)
    sh1 = jnp.pad(amax1, ((0, 0), (0, 8 - h_)))
    shg1 = jnp.take(sh1, dst, axis=0)

    vals1 = _edge_b(gs1, al1, shg1, es8, msk8, be, hc)
    s1 = jax.ops.segment_sum(vals1, dst, num_segments=n)

    wcat2 = jnp.concatenate([c2_Wl, e2_W1[:hc], c2_Wr, e2_W1[hc:2 * hc]], 1)
    bcat2 = jnp.concatenate([c2_bl, jnp.zeros((e2_W1.shape[1],), F32),
                             c2_br, jnp.zeros((e2_W1.shape[1],), F32)])
    t2 = _node2(s1, es8, c1_bias, bn_n_g, bn_n_b, bn_n_rm, bn_n_rv,
                wcat2, bcat2, bn, hc)
    w2 = hc + e2_W1.shape[1]
    gs2 = jnp.take(t2[:, :w2], src, axis=0)
    gd2 = jnp.take(t2[:, w2:], dst, axis=0)

    al2 = _edge_a2(gs2, gd2, ea1, bn_e_g, bn_e_b, bn_e_rm, bn_e_rv,
                   e2_W1[2 * hc:], e2_b1, e2_W2, e2_b2, c2_We, atts2, be, hc)

    amax2 = jax.ops.segment_max(al2[:, :h_], dst, num_segments=n)
    amax2 = jnp.where(jnp.isfinite(amax2), amax2, 0.0)
    sh2 = jnp.pad(amax2, ((0, 0), (0, 8 - h_)))
    shg2 = jnp.take(sh2, dst, axis=0)

    vals2 = _edge_b(gs2, al2, shg2, es8, msk8, be, hc)
    s2 = jax.ops.segment_sum(vals2, dst, num_segments=n)

    batch3 = batch.astype(jnp.int32).reshape(n // bn, 1, bn)
    fc2p = jnp.pad(fc2_W, ((0, 0), (0, 8 - fc2_W.shape[1])))
    fb2p = jnp.pad(fc2_b, ((0, 8 - fc2_b.shape[0]),))
    out8 = _pool(s2, batch3, es8, c2_bias, fc1_W, fc1_b, fc2p, fb2p,
                 bn, hc, ng)
    return out8[:, :fc2_W.shape[1]]


def _build_atts(att, h_, c_):
    a = jnp.zeros((8, h_ * c_), F32)
    for h in range(h_):
        a = a.at[h, h * c_:(h + 1) * c_].set(att[h])
    return a.T


# speedup vs baseline: 9.0697x; 1.3349x over previous
import functools
import jax
import jax.numpy as jnp
from jax import lax
from jax.experimental import pallas as pl
from jax.experimental.pallas import tpu as pltpu
from jax.experimental.pallas import tpu_sc as plsc

F32 = jnp.float32

_SC_NC = 2
_SC_NS = 16
_SC_NW = _SC_NC * _SC_NS


def _sc_gather(table, idx, ch):
    n, d = table.shape
    e = idx.shape[0]
    per_w = e // _SC_NW
    n_chunks = per_w // ch
    mesh = plsc.VectorSubcoreMesh(core_axis_name="c", subcore_axis_name="s")

    @functools.partial(
        pl.kernel,
        mesh=mesh,
        out_type=jax.ShapeDtypeStruct((e, d), F32),
        scratch_types=[
            pltpu.VMEM((ch,), jnp.int32),
            pltpu.VMEM((ch, d), F32),
            pltpu.SemaphoreType.DMA,
        ],
    )
    def k(table_hbm, idx_hbm, out_hbm, idx_v, rows_v, sem):
        wid = lax.axis_index("s") * _SC_NC + lax.axis_index("c")
        base = wid * per_w

        def body(j, carry):
            off = base + j * ch
            pltpu.sync_copy(idx_hbm.at[pl.ds(off, ch)], idx_v)
            pltpu.async_copy(table_hbm.at[idx_v], rows_v, sem).wait()
            pltpu.sync_copy(rows_v, out_hbm.at[pl.ds(off, ch)])
            return carry

        lax.fori_loop(0, n_chunks, body, 0)

    return k(table, idx)


def _mm(a, b):
    return jnp.dot(a, b, preferred_element_type=F32)


def _leaky(v):
    return jnp.where(v > 0, v, 0.2 * v)


def _nodeproj_body(x_ref, w_ref, b_ref, o_ref):
    o_ref[...] = _mm(x_ref[...], w_ref[...]) + b_ref[...]


def _node_proj(x, w, b, bn):
    n, k = x.shape
    m = w.shape[1]
    grid = (n // bn,)
    return pl.pallas_call(
        _nodeproj_body,
        grid=grid,
        in_specs=[
            pl.BlockSpec((bn, k), lambda i: (i, 0)),
            pl.BlockSpec((k, m), lambda i: (0, 0)),
            pl.BlockSpec((1, m), lambda i: (0, 0)),
        ],
        out_specs=pl.BlockSpec((bn, m), lambda i: (i, 0)),
        out_shape=jax.ShapeDtypeStruct((n, m), F32),
    )(x, w, b.reshape(1, m))


def _edgeA1_body(gs_ref, gd_ref, ea_ref, w1c_ref, b1_ref, w2_ref, b2_ref,
                 we_ref, atts_ref, ea1_ref, al_ref, *, hc, d1):
    g = jnp.maximum(gs_ref[:, hc:hc + d1] + gd_ref[:, hc:hc + d1]
                    + _mm(ea_ref[...], w1c_ref[...]) + b1_ref[...], 0.0)
    ea1 = _mm(g, w2_ref[...]) + b2_ref[...]
    ea1_ref[...] = ea1
    m = _leaky(gs_ref[:, :hc] + gd_ref[:, :hc] + _mm(ea1, we_ref[...]))
    al_ref[...] = _mm(m, atts_ref[...])


def _edge_a1(gs, gd, eattr, w1c, b1, w2, b2, we, atts, be, hc):
    e = gs.shape[0]
    wgs = gs.shape[1]
    de = eattr.shape[1]
    d1 = w2.shape[0]
    d2 = w2.shape[1]
    grid = (e // be,)
    return pl.pallas_call(
        functools.partial(_edgeA1_body, hc=hc, d1=d1),
        grid=grid,
        in_specs=[
            pl.BlockSpec((be, wgs), lambda i: (i, 0)),
            pl.BlockSpec((be, wgs), lambda i: (i, 0)),
            pl.BlockSpec((be, de), lambda i: (i, 0)),
            pl.BlockSpec((de, d1), lambda i: (0, 0)),
            pl.BlockSpec((1, d1), lambda i: (0, 0)),
            pl.BlockSpec((d1, d2), lambda i: (0, 0)),
            pl.BlockSpec((1, d2), lambda i: (0, 0)),
            pl.BlockSpec((d2, hc), lambda i: (0, 0)),
            pl.BlockSpec((hc, 8), lambda i: (0, 0)),
        ],
        out_specs=[
            pl.BlockSpec((be, d2), lambda i: (i, 0)),
            pl.BlockSpec((be, 8), lambda i: (i, 0)),
        ],
        out_shape=[
            jax.ShapeDtypeStruct((e, d2), F32),
            jax.ShapeDtypeStruct((e, 8), F32),
        ],
    )(gs, gd, eattr, w1c, b1.reshape(1, d1), w2, b2.reshape(1, d2), we, atts)


def _edgeA2_body(gs_ref, gd_ref, ea1_ref, bng_ref, bnb_ref, bnm_ref, bnv_ref,
                 w1c_ref, b1_ref, w2_ref, b2_ref, we_ref, atts_ref, al_ref,
                 *, hc):
    eb = ((ea1_ref[...] - bnm_ref[...])
          * jax.lax.rsqrt(bnv_ref[...] + 1e-5) * bng_ref[...] + bnb_ref[...])
    g = jnp.maximum(gs_ref[:, hc:] + gd_ref[:, hc:]
                    + _mm(eb, w1c_ref[...]) + b1_ref[...], 0.0)
    ea2 = _mm(g, w2_ref[...]) + b2_ref[...]
    m = _leaky(gs_ref[:, :hc] + gd_ref[:, :hc] + _mm(ea2, we_ref[...]))
    al_ref[...] = _mm(m, atts_ref[...])


def _edge_a2(gs, gd, ea1, bng, bnb, bnm, bnv, w1c, b1, w2, b2, we, atts,
             be, hc):
    e = gs.shape[0]
    wgs = gs.shape[1]
    d0 = ea1.shape[1]
    d1 = w2.shape[0]
    d2 = w2.shape[1]
    grid = (e // be,)
    row = lambda a: a.reshape(1, -1)
    return pl.pallas_call(
        functools.partial(_edgeA2_body, hc=hc),
        grid=grid,
        in_specs=[
            pl.BlockSpec((be, wgs), lambda i: (i, 0)),
            pl.BlockSpec((be, wgs), lambda i: (i, 0)),
            pl.BlockSpec((be, d0), lambda i: (i, 0)),
            pl.BlockSpec((1, d0), lambda i: (0, 0)),
            pl.BlockSpec((1, d0), lambda i: (0, 0)),
            pl.BlockSpec((1, d0), lambda i: (0, 0)),
            pl.BlockSpec((1, d0), lambda i: (0, 0)),
            pl.BlockSpec((d0, d1), lambda i: (0, 0)),
            pl.BlockSpec((1, d1), lambda i: (0, 0)),
            pl.BlockSpec((d1, d2), lambda i: (0, 0)),
            pl.BlockSpec((1, d2), lambda i: (0, 0)),
            pl.BlockSpec((d2, hc), lambda i: (0, 0)),
            pl.BlockSpec((hc, 8), lambda i: (0, 0)),
        ],
        out_specs=pl.BlockSpec((be, 8), lambda i: (i, 0)),
        out_shape=jax.ShapeDtypeStruct((e, 8), F32),
    )(gs, gd, ea1, row(bng), row(bnb), row(bnm), row(bnv),
      w1c, row(b1), w2, row(b2), we, atts)


def _edgeB_body(gs_ref, al_ref, sh_ref, es_ref, msk_ref, out_ref, *, hc):
    w8 = jnp.exp(al_ref[...] - sh_ref[...]) * msk_ref[...]
    wexp = _mm(w8, es_ref[...])
    out_ref[:, :hc] = gs_ref[:, :hc] * wexp
    out_ref[:, hc:] = w8


def _edge_b(gs, al8, sh8, es8, msk8, be, hc):
    e = gs.shape[0]
    wgs = gs.shape[1]
    grid = (e // be,)
    return pl.pallas_call(
        functools.partial(_edgeB_body, hc=hc),
        grid=grid,
        in_specs=[
            pl.BlockSpec((be, wgs), lambda i: (i, 0)),
            pl.BlockSpec((be, 8), lambda i: (i, 0)),
            pl.BlockSpec((be, 8), lambda i: (i, 0)),
            pl.BlockSpec((8, hc), lambda i: (0, 0)),
            pl.BlockSpec((1, 8), lambda i: (0, 0)),
        ],
        out_specs=pl.BlockSpec((be, hc + 8), lambda i: (i, 0)),
        out_shape=jax.ShapeDtypeStruct((e, hc + 8), F32),
    )(gs, al8, sh8, es8, msk8)


def _node2_body(s_ref, es_ref, bias_ref, g_ref, b_ref, m_ref, v_ref,
                w_ref, bc_ref, t2_ref, *, hc):
    den = _mm(s_ref[:, hc:], es_ref[...])
    h = jnp.maximum(s_ref[:, :hc] / (den + 1e-16) + bias_ref[...], 0.0)
    h = (h - m_ref[...]) * jax.lax.rsqrt(v_ref[...] + 1e-5) * g_ref[...] \
        + b_ref[...]
    t2_ref[...] = _mm(h, w_ref[...]) + bc_ref[...]


def _node2(s, es8, bias, g, b, m, v, wcat, bcat, bn, hc):
    n = s.shape[0]
    w2 = wcat.shape[1]
    grid = (n // bn,)
    row = lambda a: a.reshape(1, -1)
    return pl.pallas_call(
        functools.partial(_node2_body, hc=hc),
        grid=grid,
        in_specs=[
            pl.BlockSpec((bn, hc + 8), lambda i: (i, 0)),
            pl.BlockSpec((8, hc), lambda i: (0, 0)),
            pl.BlockSpec((1, hc), lambda i: (0, 0)),
            pl.BlockSpec((1, hc), lambda i: (0, 0)),
            pl.BlockSpec((1, hc), lambda i: (0, 0)),
            pl.BlockSpec((1, hc), lambda i: (0, 0)),
            pl.BlockSpec((1, hc), lambda i: (0, 0)),
            pl.BlockSpec((hc, w2), lambda i: (0, 0)),
            pl.BlockSpec((1, w2), lambda i: (0, 0)),
        ],
        out_specs=pl.BlockSpec((bn, w2), lambda i: (i, 0)),
        out_shape=jax.ShapeDtypeStruct((n, w2), F32),
    )(s, es8, row(bias), row(g), row(b), row(m), row(v), wcat, row(bcat))


def _pool_body(s_ref, bt_ref, es_ref, bias_ref, w1_ref, b1_ref, w2_ref,
               b2_ref, o_ref, acc, *, hc, ng, nblk):
    i = pl.program_id(0)

    @pl.when(i == 0)
    def _():
        acc[...] = jnp.zeros_like(acc)

    den = _mm(s_ref[:, hc:], es_ref[...])
    h = jnp.maximum(s_ref[:, :hc] / (den + 1e-16) + bias_ref[...], 0.0)
    seg = bt_ref[0]
    ids = jax.lax.broadcasted_iota(jnp.int32, (ng, seg.shape[1]), 0)
    onehot = (ids == seg).astype(F32)
    acc[...] += _mm(onehot, h)

    @pl.when(i == nblk - 1)
    def _():
        f = jnp.maximum(_mm(acc[...], w1_ref[...]) + b1_ref[...], 0.0)
        o_ref[...] = _mm(f, w2_ref[...]) + b2_ref[...]


def _pool(s, batch3, es8, bias, w1, b1, w2p, b2p, bn, hc, ng):
    n = s.shape[0]
    nblk = n // bn
    dh = w1.shape[1]
    row = lambda a: a.reshape(1, -1)
    return pl.pallas_call(
        functools.partial(_pool_body, hc=hc, ng=ng, nblk=nblk),
        grid=(nblk,),
        in_specs=[
            pl.BlockSpec((bn, hc + 8), lambda i: (i, 0)),
            pl.BlockSpec((1, 1, bn), lambda i: (i, 0, 0)),
            pl.BlockSpec((8, hc), lambda i: (0, 0)),
            pl.BlockSpec((1, hc), lambda i: (0, 0)),
            pl.BlockSpec((hc, dh), lambda i: (0, 0)),
            pl.BlockSpec((1, dh), lambda i: (0, 0)),
            pl.BlockSpec((dh, 8), lambda i: (0, 0)),
            pl.BlockSpec((1, 8), lambda i: (0, 0)),
        ],
        out_specs=pl.BlockSpec((ng, 8), lambda i: (0, 0)),
        out_shape=jax.ShapeDtypeStruct((ng, 8), F32),
        scratch_shapes=[pltpu.VMEM((ng, hc), F32)],
    )(s, batch3, es8, row(bias), w1, row(b1), w2p, row(b2p))


def kernel(x, edge_attr, e1_W1, e1_b1, e1_W2, e1_b2, c1_Wl, c1_bl, c1_Wr,
           c1_br, c1_We, c1_att, c1_bias, bn_n_g, bn_n_b, bn_n_rm, bn_n_rv,
           bn_e_g, bn_e_b, bn_e_rm, bn_e_rv, e2_W1, e2_b1, e2_W2, e2_b2,
           c2_Wl, c2_bl, c2_Wr, c2_br, c2_We, c2_att, c2_bias, fc1_W, fc1_b,
           fc2_W, fc2_b, edge_index, batch):
    n, df = x.shape
    e, de = edge_attr.shape
    h_, c_ = c1_att.shape
    hc = h_ * c_
    ng = 64
    be = 2000
    bn = 1000

    src = edge_index[0]
    dst = edge_index[1]

    eye_h = jnp.kron(jnp.eye(h_, dtype=F32), jnp.ones((1, c_), F32))
    es8 = jnp.concatenate([eye_h, jnp.zeros((8 - h_, hc), F32)], 0)
    msk8 = jnp.concatenate([jnp.ones((h_,), F32),
                            jnp.zeros((8 - h_,), F32)]).reshape(1, 8)

    atts1 = _build_atts(c1_att, h_, c_)
    atts2 = _build_atts(c2_att, h_, c_)

    d1e = e1_W1.shape[1]
    pad1 = (-(hc + d1e)) % 128
    zc = jnp.zeros((df, pad1), F32)
    zb = jnp.zeros((pad1,), F32)
    wcat1 = jnp.concatenate(
        [c1_Wl, e1_W1[:df], zc, c1_Wr, e1_W1[df:2 * df], zc], 1)
    bcat1 = jnp.concatenate(
        [c1_bl, jnp.zeros((d1e,), F32), zb,
         c1_br, jnp.zeros((d1e,), F32), zb])
    t1 = _node_proj(x, wcat1, bcat1, bn)
    w1 = hc + d1e + pad1
    src32 = src.astype(jnp.int32)
    dst32 = dst.astype(jnp.int32)
    gs1 = _sc_gather(t1[:, :w1], src32, 80)
    gd1 = _sc_gather(t1[:, w1:], dst32, 80)

    ea1, al1 = _edge_a1(gs1, gd1, edge_attr, e1_W1[2 * df:], e1_b1,
                        e1_W2, e1_b2, c1_We, atts1, be, hc)

    amax1 = jax.ops.segment_max(al1[:, :h_], dst, num_segments=n)
    amax1 = jnp.where(jnp.isfinite(amax1), amax1, 0.0)
    sh1 = jnp.pad(amax1, ((0, 0), (0, 8 - h_)))
    shg1 = jnp.take(sh1, dst, axis=0)

    vals1 = _edge_b(gs1, al1, shg1, es8, msk8, be, hc)
    s1 = jax.ops.segment_sum(vals1, dst, num_segments=n)

    wcat2 = jnp.concatenate([c2_Wl, e2_W1[:hc], c2_Wr, e2_W1[hc:2 * hc]], 1)
    bcat2 = jnp.concatenate([c2_bl, jnp.zeros((e2_W1.shape[1],), F32),
                             c2_br, jnp.zeros((e2_W1.shape[1],), F32)])
    t2 = _node2(s1, es8, c1_bias, bn_n_g, bn_n_b, bn_n_rm, bn_n_rv,
                wcat2, bcat2, bn, hc)
    w2 = hc + e2_W1.shape[1]
    gs2 = _sc_gather(t2[:, :w2], src32, 80)
    gd2 = _sc_gather(t2[:, w2:], dst32, 80)

    al2 = _edge_a2(gs2, gd2, ea1, bn_e_g, bn_e_b, bn_e_rm, bn_e_rv,
                   e2_W1[2 * hc:], e2_b1, e2_W2, e2_b2, c2_We, atts2, be, hc)

    amax2 = jax.ops.segment_max(al2[:, :h_], dst, num_segments=n)
    amax2 = jnp.where(jnp.isfinite(amax2), amax2, 0.0)
    sh2 = jnp.pad(amax2, ((0, 0), (0, 8 - h_)))
    shg2 = jnp.take(sh2, dst, axis=0)

    vals2 = _edge_b(gs2, al2, shg2, es8, msk8, be, hc)
    s2 = jax.ops.segment_sum(vals2, dst, num_segments=n)

    batch3 = batch.astype(jnp.int32).reshape(n // bn, 1, bn)
    fc2p = jnp.pad(fc2_W, ((0, 0), (0, 8 - fc2_W.shape[1])))
    fb2p = jnp.pad(fc2_b, ((0, 8 - fc2_b.shape[0]),))
    out8 = _pool(s2, batch3, es8, c2_bias, fc1_W, fc1_b, fc2p, fb2p,
                 bn, hc, ng)
    return out8[:, :fc2_W.shape[1]]


def _build_atts(att, h_, c_):
    a = jnp.zeros((8, h_ * c_), F32)
    for h in range(h_):
        a = a.at[h, h * c_:(h + 1) * c_].set(att[h])
    return a.T


# speedup vs baseline: 9.3040x; 1.0258x over previous
import functools
import jax
import jax.numpy as jnp
from jax import lax
from jax.experimental import pallas as pl
from jax.experimental.pallas import tpu as pltpu
from jax.experimental.pallas import tpu_sc as plsc

F32 = jnp.float32

_SC_NC = 2
_SC_NS = 16
_SC_NW = _SC_NC * _SC_NS


def _sc_gather(table, idx, ch):
    n, d = table.shape
    e = idx.shape[0]
    per_w = e // _SC_NW
    n_chunks = per_w // ch
    tail = per_w - n_chunks * ch
    mesh = plsc.VectorSubcoreMesh(core_axis_name="c", subcore_axis_name="s")

    @functools.partial(
        pl.kernel,
        mesh=mesh,
        out_type=jax.ShapeDtypeStruct((e, d), F32),
        scratch_types=[
            pltpu.VMEM((ch,), jnp.int32),
            pltpu.VMEM((ch, d), F32),
            pltpu.SemaphoreType.DMA,
        ],
    )
    def k(table_hbm, idx_hbm, out_hbm, idx_v, rows_v, sem):
        wid = lax.axis_index("s") * _SC_NC + lax.axis_index("c")
        base = wid * per_w

        def body(j, carry):
            off = base + j * ch
            pltpu.sync_copy(idx_hbm.at[pl.ds(off, ch)], idx_v)
            pltpu.async_copy(table_hbm.at[idx_v], rows_v, sem).wait()
            pltpu.sync_copy(rows_v, out_hbm.at[pl.ds(off, ch)])
            return carry

        lax.fori_loop(0, n_chunks, body, 0)
        if tail:
            off = base + n_chunks * ch
            pltpu.sync_copy(idx_hbm.at[pl.ds(off, tail)],
                            idx_v.at[pl.ds(0, tail)])
            pltpu.async_copy(table_hbm.at[idx_v.at[pl.ds(0, tail)]],
                             rows_v.at[pl.ds(0, tail)], sem).wait()
            pltpu.sync_copy(rows_v.at[pl.ds(0, tail)],
                            out_hbm.at[pl.ds(off, tail)])

    return k(table, idx)


def _mm(a, b):
    return jnp.dot(a, b, preferred_element_type=F32)


def _leaky(v):
    return jnp.where(v > 0, v, 0.2 * v)


def _nodeproj_body(x_ref, w_ref, b_ref, o_ref):
    o_ref[...] = _mm(x_ref[...], w_ref[...]) + b_ref[...]


def _node_proj(x, w, b, bn):
    n, k = x.shape
    m = w.shape[1]
    grid = (n // bn,)
    return pl.pallas_call(
        _nodeproj_body,
        grid=grid,
        in_specs=[
            pl.BlockSpec((bn, k), lambda i: (i, 0)),
            pl.BlockSpec((k, m), lambda i: (0, 0)),
            pl.BlockSpec((1, m), lambda i: (0, 0)),
        ],
        out_specs=pl.BlockSpec((bn, m), lambda i: (i, 0)),
        out_shape=jax.ShapeDtypeStruct((n, m), F32),
    )(x, w, b.reshape(1, m))


def _edgeA1_body(gs_ref, gd_ref, ea_ref, w1c_ref, b1_ref, w2_ref, b2_ref,
                 we_ref, atts_ref, ea1_ref, al_ref, *, hc, d1):
    g = jnp.maximum(gs_ref[:, hc:hc + d1] + gd_ref[:, hc:hc + d1]
                    + _mm(ea_ref[...], w1c_ref[...]) + b1_ref[...], 0.0)
    ea1 = _mm(g, w2_ref[...]) + b2_ref[...]
    ea1_ref[...] = ea1
    m = _leaky(gs_ref[:, :hc] + gd_ref[:, :hc] + _mm(ea1, we_ref[...]))
    al_ref[...] = _mm(m, atts_ref[...])


def _edge_a1(gs, gd, eattr, w1c, b1, w2, b2, we, atts, be, hc):
    e = gs.shape[0]
    wgs = gs.shape[1]
    de = eattr.shape[1]
    d1 = w2.shape[0]
    d2 = w2.shape[1]
    grid = (e // be,)
    return pl.pallas_call(
        functools.partial(_edgeA1_body, hc=hc, d1=d1),
        grid=grid,
        in_specs=[
            pl.BlockSpec((be, wgs), lambda i: (i, 0)),
            pl.BlockSpec((be, wgs), lambda i: (i, 0)),
            pl.BlockSpec((be, de), lambda i: (i, 0)),
            pl.BlockSpec((de, d1), lambda i: (0, 0)),
            pl.BlockSpec((1, d1), lambda i: (0, 0)),
            pl.BlockSpec((d1, d2), lambda i: (0, 0)),
            pl.BlockSpec((1, d2), lambda i: (0, 0)),
            pl.BlockSpec((d2, hc), lambda i: (0, 0)),
            pl.BlockSpec((hc, 8), lambda i: (0, 0)),
        ],
        out_specs=[
            pl.BlockSpec((be, d2), lambda i: (i, 0)),
            pl.BlockSpec((be, 8), lambda i: (i, 0)),
        ],
        out_shape=[
            jax.ShapeDtypeStruct((e, d2), F32),
            jax.ShapeDtypeStruct((e, 8), F32),
        ],
    )(gs, gd, eattr, w1c, b1.reshape(1, d1), w2, b2.reshape(1, d2), we, atts)


def _edgeA2_body(gs_ref, gd_ref, ea1_ref, bng_ref, bnb_ref, bnm_ref, bnv_ref,
                 w1c_ref, b1_ref, w2_ref, b2_ref, we_ref, atts_ref, al_ref,
                 *, hc):
    eb = ((ea1_ref[...] - bnm_ref[...])
          * jax.lax.rsqrt(bnv_ref[...] + 1e-5) * bng_ref[...] + bnb_ref[...])
    g = jnp.maximum(gs_ref[:, hc:] + gd_ref[:, hc:]
                    + _mm(eb, w1c_ref[...]) + b1_ref[...], 0.0)
    ea2 = _mm(g, w2_ref[...]) + b2_ref[...]
    m = _leaky(gs_ref[:, :hc] + gd_ref[:, :hc] + _mm(ea2, we_ref[...]))
    al_ref[...] = _mm(m, atts_ref[...])


def _edge_a2(gs, gd, ea1, bng, bnb, bnm, bnv, w1c, b1, w2, b2, we, atts,
             be, hc):
    e = gs.shape[0]
    wgs = gs.shape[1]
    d0 = ea1.shape[1]
    d1 = w2.shape[0]
    d2 = w2.shape[1]
    grid = (e // be,)
    row = lambda a: a.reshape(1, -1)
    return pl.pallas_call(
        functools.partial(_edgeA2_body, hc=hc),
        grid=grid,
        in_specs=[
            pl.BlockSpec((be, wgs), lambda i: (i, 0)),
            pl.BlockSpec((be, wgs), lambda i: (i, 0)),
            pl.BlockSpec((be, d0), lambda i: (i, 0)),
            pl.BlockSpec((1, d0), lambda i: (0, 0)),
            pl.BlockSpec((1, d0), lambda i: (0, 0)),
            pl.BlockSpec((1, d0), lambda i: (0, 0)),
            pl.BlockSpec((1, d0), lambda i: (0, 0)),
            pl.BlockSpec((d0, d1), lambda i: (0, 0)),
            pl.BlockSpec((1, d1), lambda i: (0, 0)),
            pl.BlockSpec((d1, d2), lambda i: (0, 0)),
            pl.BlockSpec((1, d2), lambda i: (0, 0)),
            pl.BlockSpec((d2, hc), lambda i: (0, 0)),
            pl.BlockSpec((hc, 8), lambda i: (0, 0)),
        ],
        out_specs=pl.BlockSpec((be, 8), lambda i: (i, 0)),
        out_shape=jax.ShapeDtypeStruct((e, 8), F32),
    )(gs, gd, ea1, row(bng), row(bnb), row(bnm), row(bnv),
      w1c, row(b1), w2, row(b2), we, atts)


def _edgeB_body(gs_ref, al_ref, sh_ref, es_ref, msk_ref, out_ref, *, hc):
    w8 = jnp.exp(al_ref[...] - sh_ref[...]) * msk_ref[...]
    wexp = _mm(w8, es_ref[...])
    out_ref[:, :hc] = gs_ref[:, :hc] * wexp
    out_ref[:, hc:] = w8


def _edge_b(gs, al8, sh8, es8, msk8, be, hc):
    e = gs.shape[0]
    wgs = gs.shape[1]
    grid = (e // be,)
    return pl.pallas_call(
        functools.partial(_edgeB_body, hc=hc),
        grid=grid,
        in_specs=[
            pl.BlockSpec((be, wgs), lambda i: (i, 0)),
            pl.BlockSpec((be, 8), lambda i: (i, 0)),
            pl.BlockSpec((be, 8), lambda i: (i, 0)),
            pl.BlockSpec((8, hc), lambda i: (0, 0)),
            pl.BlockSpec((1, 8), lambda i: (0, 0)),
        ],
        out_specs=pl.BlockSpec((be, hc + 8), lambda i: (i, 0)),
        out_shape=jax.ShapeDtypeStruct((e, hc + 8), F32),
    )(gs, al8, sh8, es8, msk8)


def _node2_body(s_ref, es_ref, bias_ref, g_ref, b_ref, m_ref, v_ref,
                w_ref, bc_ref, t2_ref, *, hc):
    den = _mm(s_ref[:, hc:], es_ref[...])
    h = jnp.maximum(s_ref[:, :hc] / (den + 1e-16) + bias_ref[...], 0.0)
    h = (h - m_ref[...]) * jax.lax.rsqrt(v_ref[...] + 1e-5) * g_ref[...] \
        + b_ref[...]
    t2_ref[...] = _mm(h, w_ref[...]) + bc_ref[...]


def _node2(s, es8, bias, g, b, m, v, wcat, bcat, bn, hc):
    n = s.shape[0]
    w2 = wcat.shape[1]
    grid = (n // bn,)
    row = lambda a: a.reshape(1, -1)
    return pl.pallas_call(
        functools.partial(_node2_body, hc=hc),
        grid=grid,
        in_specs=[
            pl.BlockSpec((bn, hc + 8), lambda i: (i, 0)),
            pl.BlockSpec((8, hc), lambda i: (0, 0)),
            pl.BlockSpec((1, hc), lambda i: (0, 0)),
            pl.BlockSpec((1, hc), lambda i: (0, 0)),
            pl.BlockSpec((1, hc), lambda i: (0, 0)),
            pl.BlockSpec((1, hc), lambda i: (0, 0)),
            pl.BlockSpec((1, hc), lambda i: (0, 0)),
            pl.BlockSpec((hc, w2), lambda i: (0, 0)),
            pl.BlockSpec((1, w2), lambda i: (0, 0)),
        ],
        out_specs=pl.BlockSpec((bn, w2), lambda i: (i, 0)),
        out_shape=jax.ShapeDtypeStruct((n, w2), F32),
    )(s, es8, row(bias), row(g), row(b), row(m), row(v), wcat, row(bcat))


def _pool_body(s_ref, bt_ref, es_ref, bias_ref, w1_ref, b1_ref, w2_ref,
               b2_ref, o_ref, acc, *, hc, ng, nblk):
    i = pl.program_id(0)

    @pl.when(i == 0)
    def _():
        acc[...] = jnp.zeros_like(acc)

    den = _mm(s_ref[:, hc:], es_ref[...])
    h = jnp.maximum(s_ref[:, :hc] / (den + 1e-16) + bias_ref[...], 0.0)
    seg = bt_ref[0]
    ids = jax.lax.broadcasted_iota(jnp.int32, (ng, seg.shape[1]), 0)
    onehot = (ids == seg).astype(F32)
    acc[...] += _mm(onehot, h)

    @pl.when(i == nblk - 1)
    def _():
        f = jnp.maximum(_mm(acc[...], w1_ref[...]) + b1_ref[...], 0.0)
        o_ref[...] = _mm(f, w2_ref[...]) + b2_ref[...]


def _pool(s, batch3, es8, bias, w1, b1, w2p, b2p, bn, hc, ng):
    n = s.shape[0]
    nblk = n // bn
    dh = w1.shape[1]
    row = lambda a: a.reshape(1, -1)
    return pl.pallas_call(
        functools.partial(_pool_body, hc=hc, ng=ng, nblk=nblk),
        grid=(nblk,),
        in_specs=[
            pl.BlockSpec((bn, hc + 8), lambda i: (i, 0)),
            pl.BlockSpec((1, 1, bn), lambda i: (i, 0, 0)),
            pl.BlockSpec((8, hc), lambda i: (0, 0)),
            pl.BlockSpec((1, hc), lambda i: (0, 0)),
            pl.BlockSpec((hc, dh), lambda i: (0, 0)),
            pl.BlockSpec((1, dh), lambda i: (0, 0)),
            pl.BlockSpec((dh, 8), lambda i: (0, 0)),
            pl.BlockSpec((1, 8), lambda i: (0, 0)),
        ],
        out_specs=pl.BlockSpec((ng, 8), lambda i: (0, 0)),
        out_shape=jax.ShapeDtypeStruct((ng, 8), F32),
        scratch_shapes=[pltpu.VMEM((ng, hc), F32)],
    )(s, batch3, es8, row(bias), w1, row(b1), w2p, row(b2p))


def kernel(x, edge_attr, e1_W1, e1_b1, e1_W2, e1_b2, c1_Wl, c1_bl, c1_Wr,
           c1_br, c1_We, c1_att, c1_bias, bn_n_g, bn_n_b, bn_n_rm, bn_n_rv,
           bn_e_g, bn_e_b, bn_e_rm, bn_e_rv, e2_W1, e2_b1, e2_W2, e2_b2,
           c2_Wl, c2_bl, c2_Wr, c2_br, c2_We, c2_att, c2_bias, fc1_W, fc1_b,
           fc2_W, fc2_b, edge_index, batch):
    n, df = x.shape
    e, de = edge_attr.shape
    h_, c_ = c1_att.shape
    hc = h_ * c_
    ng = 64
    be = 2000
    bn = 1000

    src = edge_index[0]
    dst = edge_index[1]

    eye_h = jnp.kron(jnp.eye(h_, dtype=F32), jnp.ones((1, c_), F32))
    es8 = jnp.concatenate([eye_h, jnp.zeros((8 - h_, hc), F32)], 0)
    msk8 = jnp.concatenate([jnp.ones((h_,), F32),
                            jnp.zeros((8 - h_,), F32)]).reshape(1, 8)

    atts1 = _build_atts(c1_att, h_, c_)
    atts2 = _build_atts(c2_att, h_, c_)

    d1e = e1_W1.shape[1]
    pad1 = (-(hc + d1e)) % 128
    zc = jnp.zeros((df, pad1), F32)
    zb = jnp.zeros((pad1,), F32)
    wcat1 = jnp.concatenate(
        [c1_Wl, e1_W1[:df], zc, c1_Wr, e1_W1[df:2 * df], zc], 1)
    bcat1 = jnp.concatenate(
        [c1_bl, jnp.zeros((d1e,), F32), zb,
         c1_br, jnp.zeros((d1e,), F32), zb])
    t1 = _node_proj(x, wcat1, bcat1, bn)
    w1 = hc + d1e + pad1
    src32 = src.astype(jnp.int32)
    dst32 = dst.astype(jnp.int32)
    gs1 = _sc_gather(t1[:, :w1], src32, 128)
    gd1 = _sc_gather(t1[:, w1:], dst32, 128)

    ea1, al1 = _edge_a1(gs1, gd1, edge_attr, e1_W1[2 * df:], e1_b1,
                        e1_W2, e1_b2, c1_We, atts1, be, hc)

    amax1 = jax.ops.segment_max(al1[:, :h_], dst, num_segments=n)
    amax1 = jnp.where(jnp.isfinite(amax1), amax1, 0.0)
    sh1 = jnp.pad(amax1, ((0, 0), (0, 8 - h_)))
    shg1 = jnp.take(sh1, dst, axis=0)

    vals1 = _edge_b(gs1, al1, shg1, es8, msk8, be, hc)
    s1 = jax.ops.segment_sum(vals1, dst, num_segments=n)

    wcat2 = jnp.concatenate([c2_Wl, e2_W1[:hc], c2_Wr, e2_W1[hc:2 * hc]], 1)
    bcat2 = jnp.concatenate([c2_bl, jnp.zeros((e2_W1.shape[1],), F32),
                             c2_br, jnp.zeros((e2_W1.shape[1],), F32)])
    t2 = _node2(s1, es8, c1_bias, bn_n_g, bn_n_b, bn_n_rm, bn_n_rv,
                wcat2, bcat2, bn, hc)
    w2 = hc + e2_W1.shape[1]
    gs2 = _sc_gather(t2[:, :w2], src32, 128)
    gd2 = _sc_gather(t2[:, w2:], dst32, 128)

    al2 = _edge_a2(gs2, gd2, ea1, bn_e_g, bn_e_b, bn_e_rm, bn_e_rv,
                   e2_W1[2 * hc:], e2_b1, e2_W2, e2_b2, c2_We, atts2, be, hc)

    amax2 = jax.ops.segment_max(al2[:, :h_], dst, num_segments=n)
    amax2 = jnp.where(jnp.isfinite(amax2), amax2, 0.0)
    sh2 = jnp.pad(amax2, ((0, 0), (0, 8 - h_)))
    shg2 = jnp.take(sh2, dst, axis=0)

    vals2 = _edge_b(gs2, al2, shg2, es8, msk8, be, hc)
    s2 = jax.ops.segment_sum(vals2, dst, num_segments=n)

    batch3 = batch.astype(jnp.int32).reshape(n // bn, 1, bn)
    fc2p = jnp.pad(fc2_W, ((0, 0), (0, 8 - fc2_W.shape[1])))
    fb2p = jnp.pad(fc2_b, ((0, 8 - fc2_b.shape[0]),))
    out8 = _pool(s2, batch3, es8, c2_bias, fc1_W, fc1_b, fc2p, fb2p,
                 bn, hc, ng)
    return out8[:, :fc2_W.shape[1]]


def _build_atts(att, h_, c_):
    a = jnp.zeros((8, h_ * c_), F32)
    for h in range(h_):
        a = a.at[h, h * c_:(h + 1) * c_].set(att[h])
    return a.T


# speedup vs baseline: 9.6606x; 1.0383x over previous
import functools
import jax
import jax.numpy as jnp
from jax import lax
from jax.experimental import pallas as pl
from jax.experimental.pallas import tpu as pltpu
from jax.experimental.pallas import tpu_sc as plsc

F32 = jnp.float32

_SC_NC = 2
_SC_NS = 16
_SC_NW = _SC_NC * _SC_NS


def _sc_gather(table, idx, ch):
    n, d = table.shape
    e = idx.shape[0]
    per_w = e // _SC_NW
    n_chunks = per_w // ch
    tail = per_w - n_chunks * ch
    mesh = plsc.VectorSubcoreMesh(core_axis_name="c", subcore_axis_name="s")

    assert n_chunks % 2 == 0

    @functools.partial(
        pl.kernel,
        mesh=mesh,
        out_type=jax.ShapeDtypeStruct((e, d), F32),
        scratch_types=[
            pltpu.VMEM((per_w,), jnp.int32),
            pltpu.VMEM((ch, d), F32),
            pltpu.VMEM((ch, d), F32),
            pltpu.SemaphoreType.DMA,
            pltpu.SemaphoreType.DMA,
        ],
    )
    def k(table_hbm, idx_hbm, out_hbm, idx_v, r0, r1, s0, s1):
        wid = lax.axis_index("s") * _SC_NC + lax.axis_index("c")
        base = wid * per_w
        pltpu.sync_copy(idx_hbm.at[pl.ds(base, per_w)], idx_v)

        def gather(j, buf, sem):
            return pltpu.async_copy(
                table_hbm.at[idx_v.at[pl.ds(j * ch, ch)]], buf, sem)

        def store(j, buf):
            pltpu.sync_copy(buf, out_hbm.at[pl.ds(base + j * ch, ch)])

        gather(0, r0, s0)

        def body(g, carry):
            j0 = 2 * g
            gather(j0 + 1, r1, s1)
            pltpu.make_async_copy(
                table_hbm.at[idx_v.at[pl.ds(j0 * ch, ch)]], r0, s0).wait()
            store(j0, r0)

            @pl.when(j0 + 2 < n_chunks)
            def _():
                gather(j0 + 2, r0, s0)

            pltpu.make_async_copy(
                table_hbm.at[idx_v.at[pl.ds((j0 + 1) * ch, ch)]],
                r1, s1).wait()
            store(j0 + 1, r1)
            return carry

        lax.fori_loop(0, n_chunks // 2, body, 0)
        if tail:
            off = n_chunks * ch
            pltpu.async_copy(
                table_hbm.at[idx_v.at[pl.ds(off, tail)]],
                r0.at[pl.ds(0, tail)], s0).wait()
            pltpu.sync_copy(r0.at[pl.ds(0, tail)],
                            out_hbm.at[pl.ds(base + off, tail)])

    return k(table, idx)


def _mm(a, b):
    return jnp.dot(a, b, preferred_element_type=F32)


def _leaky(v):
    return jnp.where(v > 0, v, 0.2 * v)


def _nodeproj_body(x_ref, w_ref, b_ref, o_ref):
    o_ref[...] = _mm(x_ref[...], w_ref[...]) + b_ref[...]


def _node_proj(x, w, b, bn):
    n, k = x.shape
    m = w.shape[1]
    grid = (n // bn,)
    return pl.pallas_call(
        _nodeproj_body,
        grid=grid,
        in_specs=[
            pl.BlockSpec((bn, k), lambda i: (i, 0)),
            pl.BlockSpec((k, m), lambda i: (0, 0)),
            pl.BlockSpec((1, m), lambda i: (0, 0)),
        ],
        out_specs=pl.BlockSpec((bn, m), lambda i: (i, 0)),
        out_shape=jax.ShapeDtypeStruct((n, m), F32),
    )(x, w, b.reshape(1, m))


def _edgeA1_body(gs_ref, gd_ref, ea_ref, w1c_ref, b1_ref, w2_ref, b2_ref,
                 we_ref, atts_ref, ea1_ref, al_ref, *, hc, d1):
    g = jnp.maximum(gs_ref[:, hc:hc + d1] + gd_ref[:, hc:hc + d1]
                    + _mm(ea_ref[...], w1c_ref[...]) + b1_ref[...], 0.0)
    ea1 = _mm(g, w2_ref[...]) + b2_ref[...]
    ea1_ref[...] = ea1
    m = _leaky(gs_ref[:, :hc] + gd_ref[:, :hc] + _mm(ea1, we_ref[...]))
    al_ref[...] = _mm(m, atts_ref[...])


def _edge_a1(gs, gd, eattr, w1c, b1, w2, b2, we, atts, be, hc):
    e = gs.shape[0]
    wgs = gs.shape[1]
    de = eattr.shape[1]
    d1 = w2.shape[0]
    d2 = w2.shape[1]
    grid = (e // be,)
    return pl.pallas_call(
        functools.partial(_edgeA1_body, hc=hc, d1=d1),
        grid=grid,
        in_specs=[
            pl.BlockSpec((be, wgs), lambda i: (i, 0)),
            pl.BlockSpec((be, wgs), lambda i: (i, 0)),
            pl.BlockSpec((be, de), lambda i: (i, 0)),
            pl.BlockSpec((de, d1), lambda i: (0, 0)),
            pl.BlockSpec((1, d1), lambda i: (0, 0)),
            pl.BlockSpec((d1, d2), lambda i: (0, 0)),
            pl.BlockSpec((1, d2), lambda i: (0, 0)),
            pl.BlockSpec((d2, hc), lambda i: (0, 0)),
            pl.BlockSpec((hc, 8), lambda i: (0, 0)),
        ],
        out_specs=[
            pl.BlockSpec((be, d2), lambda i: (i, 0)),
            pl.BlockSpec((be, 8), lambda i: (i, 0)),
        ],
        out_shape=[
            jax.ShapeDtypeStruct((e, d2), F32),
            jax.ShapeDtypeStruct((e, 8), F32),
        ],
    )(gs, gd, eattr, w1c, b1.reshape(1, d1), w2, b2.reshape(1, d2), we, atts)


def _edgeA2_body(gs_ref, gd_ref, ea1_ref, bng_ref, bnb_ref, bnm_ref, bnv_ref,
                 w1c_ref, b1_ref, w2_ref, b2_ref, we_ref, atts_ref, al_ref,
                 *, hc):
    eb = ((ea1_ref[...] - bnm_ref[...])
          * jax.lax.rsqrt(bnv_ref[...] + 1e-5) * bng_ref[...] + bnb_ref[...])
    g = jnp.maximum(gs_ref[:, hc:] + gd_ref[:, hc:]
                    + _mm(eb, w1c_ref[...]) + b1_ref[...], 0.0)
    ea2 = _mm(g, w2_ref[...]) + b2_ref[...]
    m = _leaky(gs_ref[:, :hc] + gd_ref[:, :hc] + _mm(ea2, we_ref[...]))
    al_ref[...] = _mm(m, atts_ref[...])


def _edge_a2(gs, gd, ea1, bng, bnb, bnm, bnv, w1c, b1, w2, b2, we, atts,
             be, hc):
    e = gs.shape[0]
    wgs = gs.shape[1]
    d0 = ea1.shape[1]
    d1 = w2.shape[0]
    d2 = w2.shape[1]
    grid = (e // be,)
    row = lambda a: a.reshape(1, -1)
    return pl.pallas_call(
        functools.partial(_edgeA2_body, hc=hc),
        grid=grid,
        in_specs=[
            pl.BlockSpec((be, wgs), lambda i: (i, 0)),
            pl.BlockSpec((be, wgs), lambda i: (i, 0)),
            pl.BlockSpec((be, d0), lambda i: (i, 0)),
            pl.BlockSpec((1, d0), lambda i: (0, 0)),
            pl.BlockSpec((1, d0), lambda i: (0, 0)),
            pl.BlockSpec((1, d0), lambda i: (0, 0)),
            pl.BlockSpec((1, d0), lambda i: (0, 0)),
            pl.BlockSpec((d0, d1), lambda i: (0, 0)),
            pl.BlockSpec((1, d1), lambda i: (0, 0)),
            pl.BlockSpec((d1, d2), lambda i: (0, 0)),
            pl.BlockSpec((1, d2), lambda i: (0, 0)),
            pl.BlockSpec((d2, hc), lambda i: (0, 0)),
            pl.BlockSpec((hc, 8), lambda i: (0, 0)),
        ],
        out_specs=pl.BlockSpec((be, 8), lambda i: (i, 0)),
        out_shape=jax.ShapeDtypeStruct((e, 8), F32),
    )(gs, gd, ea1, row(bng), row(bnb), row(bnm), row(bnv),
      w1c, row(b1), w2, row(b2), we, atts)


def _edgeB_body(gs_ref, al_ref, sh_ref, es_ref, msk_ref, out_ref, *, hc):
    w8 = jnp.exp(al_ref[...] - sh_ref[...]) * msk_ref[...]
    wexp = _mm(w8, es_ref[...])
    out_ref[:, :hc] = gs_ref[:, :hc] * wexp
    out_ref[:, hc:] = w8


def _edge_b(gs, al8, sh8, es8, msk8, be, hc):
    e = gs.shape[0]
    wgs = gs.shape[1]
    grid = (e // be,)
    return pl.pallas_call(
        functools.partial(_edgeB_body, hc=hc),
        grid=grid,
        in_specs=[
            pl.BlockSpec((be, wgs), lambda i: (i, 0)),
            pl.BlockSpec((be, 8), lambda i: (i, 0)),
            pl.BlockSpec((be, 8), lambda i: (i, 0)),
            pl.BlockSpec((8, hc), lambda i: (0, 0)),
            pl.BlockSpec((1, 8), lambda i: (0, 0)),
        ],
        out_specs=pl.BlockSpec((be, hc + 8), lambda i: (i, 0)),
        out_shape=jax.ShapeDtypeStruct((e, hc + 8), F32),
    )(gs, al8, sh8, es8, msk8)


def _node2_body(s_ref, es_ref, bias_ref, g_ref, b_ref, m_ref, v_ref,
                w_ref, bc_ref, t2_ref, *, hc):
    den = _mm(s_ref[:, hc:], es_ref[...])
    h = jnp.maximum(s_ref[:, :hc] / (den + 1e-16) + bias_ref[...], 0.0)
    h = (h - m_ref[...]) * jax.lax.rsqrt(v_ref[...] + 1e-5) * g_ref[...] \
        + b_ref[...]
    t2_ref[...] = _mm(h, w_ref[...]) + bc_ref[...]


def _node2(s, es8, bias, g, b, m, v, wcat, bcat, bn, hc):
    n = s.shape[0]
    w2 = wcat.shape[1]
    grid = (n // bn,)
    row = lambda a: a.reshape(1, -1)
    return pl.pallas_call(
        functools.partial(_node2_body, hc=hc),
        grid=grid,
        in_specs=[
            pl.BlockSpec((bn, hc + 8), lambda i: (i, 0)),
            pl.BlockSpec((8, hc), lambda i: (0, 0)),
            pl.BlockSpec((1, hc), lambda i: (0, 0)),
            pl.BlockSpec((1, hc), lambda i: (0, 0)),
            pl.BlockSpec((1, hc), lambda i: (0, 0)),
            pl.BlockSpec((1, hc), lambda i: (0, 0)),
            pl.BlockSpec((1, hc), lambda i: (0, 0)),
            pl.BlockSpec((hc, w2), lambda i: (0, 0)),
            pl.BlockSpec((1, w2), lambda i: (0, 0)),
        ],
        out_specs=pl.BlockSpec((bn, w2), lambda i: (i, 0)),
        out_shape=jax.ShapeDtypeStruct((n, w2), F32),
    )(s, es8, row(bias), row(g), row(b), row(m), row(v), wcat, row(bcat))


def _pool_body(s_ref, bt_ref, es_ref, bias_ref, w1_ref, b1_ref, w2_ref,
               b2_ref, o_ref, acc, *, hc, ng, nblk):
    i = pl.program_id(0)

    @pl.when(i == 0)
    def _():
        acc[...] = jnp.zeros_like(acc)

    den = _mm(s_ref[:, hc:], es_ref[...])
    h = jnp.maximum(s_ref[:, :hc] / (den + 1e-16) + bias_ref[...], 0.0)
    seg = bt_ref[0]
    ids = jax.lax.broadcasted_iota(jnp.int32, (ng, seg.shape[1]), 0)
    onehot = (ids == seg).astype(F32)
    acc[...] += _mm(onehot, h)

    @pl.when(i == nblk - 1)
    def _():
        f = jnp.maximum(_mm(acc[...], w1_ref[...]) + b1_ref[...], 0.0)
        o_ref[...] = _mm(f, w2_ref[...]) + b2_ref[...]


def _pool(s, batch3, es8, bias, w1, b1, w2p, b2p, bn, hc, ng):
    n = s.shape[0]
    nblk = n // bn
    dh = w1.shape[1]
    row = lambda a: a.reshape(1, -1)
    return pl.pallas_call(
        functools.partial(_pool_body, hc=hc, ng=ng, nblk=nblk),
        grid=(nblk,),
        in_specs=[
            pl.BlockSpec((bn, hc + 8), lambda i: (i, 0)),
            pl.BlockSpec((1, 1, bn), lambda i: (i, 0, 0)),
            pl.BlockSpec((8, hc), lambda i: (0, 0)),
            pl.BlockSpec((1, hc), lambda i: (0, 0)),
            pl.BlockSpec((hc, dh), lambda i: (0, 0)),
            pl.BlockSpec((1, dh), lambda i: (0, 0)),
            pl.BlockSpec((dh, 8), lambda i: (0, 0)),
            pl.BlockSpec((1, 8), lambda i: (0, 0)),
        ],
        out_specs=pl.BlockSpec((ng, 8), lambda i: (0, 0)),
        out_shape=jax.ShapeDtypeStruct((ng, 8), F32),
        scratch_shapes=[pltpu.VMEM((ng, hc), F32)],
    )(s, batch3, es8, row(bias), w1, row(b1), w2p, row(b2p))


def kernel(x, edge_attr, e1_W1, e1_b1, e1_W2, e1_b2, c1_Wl, c1_bl, c1_Wr,
           c1_br, c1_We, c1_att, c1_bias, bn_n_g, bn_n_b, bn_n_rm, bn_n_rv,
           bn_e_g, bn_e_b, bn_e_rm, bn_e_rv, e2_W1, e2_b1, e2_W2, e2_b2,
           c2_Wl, c2_bl, c2_Wr, c2_br, c2_We, c2_att, c2_bias, fc1_W, fc1_b,
           fc2_W, fc2_b, edge_index, batch):
    n, df = x.shape
    e, de = edge_attr.shape
    h_, c_ = c1_att.shape
    hc = h_ * c_
    ng = 64
    be = 2000
    bn = 1000

    src = edge_index[0]
    dst = edge_index[1]

    eye_h = jnp.kron(jnp.eye(h_, dtype=F32), jnp.ones((1, c_), F32))
    es8 = jnp.concatenate([eye_h, jnp.zeros((8 - h_, hc), F32)], 0)
    msk8 = jnp.concatenate([jnp.ones((h_,), F32),
                            jnp.zeros((8 - h_,), F32)]).reshape(1, 8)

    atts1 = _build_atts(c1_att, h_, c_)
    atts2 = _build_atts(c2_att, h_, c_)

    d1e = e1_W1.shape[1]
    pad1 = (-(hc + d1e)) % 128
    zc = jnp.zeros((df, pad1), F32)
    zb = jnp.zeros((pad1,), F32)
    wcat1 = jnp.concatenate(
        [c1_Wl, e1_W1[:df], zc, c1_Wr, e1_W1[df:2 * df], zc], 1)
    bcat1 = jnp.concatenate(
        [c1_bl, jnp.zeros((d1e,), F32), zb,
         c1_br, jnp.zeros((d1e,), F32), zb])
    t1 = _node_proj(x, wcat1, bcat1, bn)
    w1 = hc + d1e + pad1
    src32 = src.astype(jnp.int32)
    dst32 = dst.astype(jnp.int32)
    gs1 = _sc_gather(t1[:, :w1], src32, 128)
    gd1 = _sc_gather(t1[:, w1:], dst32, 128)

    ea1, al1 = _edge_a1(gs1, gd1, edge_attr, e1_W1[2 * df:], e1_b1,
                        e1_W2, e1_b2, c1_We, atts1, be, hc)

    amax1 = jax.ops.segment_max(al1[:, :h_], dst, num_segments=n)
    amax1 = jnp.where(jnp.isfinite(amax1), amax1, 0.0)
    sh1 = jnp.pad(amax1, ((0, 0), (0, 8 - h_)))
    shg1 = jnp.take(sh1, dst, axis=0)

    vals1 = _edge_b(gs1, al1, shg1, es8, msk8, be, hc)
    s1 = jax.ops.segment_sum(vals1, dst, num_segments=n)

    wcat2 = jnp.concatenate([c2_Wl, e2_W1[:hc], c2_Wr, e2_W1[hc:2 * hc]], 1)
    bcat2 = jnp.concatenate([c2_bl, jnp.zeros((e2_W1.shape[1],), F32),
                             c2_br, jnp.zeros((e2_W1.shape[1],), F32)])
    t2 = _node2(s1, es8, c1_bias, bn_n_g, bn_n_b, bn_n_rm, bn_n_rv,
                wcat2, bcat2, bn, hc)
    w2 = hc + e2_W1.shape[1]
    gs2 = _sc_gather(t2[:, :w2], src32, 128)
    gd2 = _sc_gather(t2[:, w2:], dst32, 128)

    al2 = _edge_a2(gs2, gd2, ea1, bn_e_g, bn_e_b, bn_e_rm, bn_e_rv,
                   e2_W1[2 * hc:], e2_b1, e2_W2, e2_b2, c2_We, atts2, be, hc)

    amax2 = jax.ops.segment_max(al2[:, :h_], dst, num_segments=n)
    amax2 = jnp.where(jnp.isfinite(amax2), amax2, 0.0)
    sh2 = jnp.pad(amax2, ((0, 0), (0, 8 - h_)))
    shg2 = jnp.take(sh2, dst, axis=0)

    vals2 = _edge_b(gs2, al2, shg2, es8, msk8, be, hc)
    s2 = jax.ops.segment_sum(vals2, dst, num_segments=n)

    batch3 = batch.astype(jnp.int32).reshape(n // bn, 1, bn)
    fc2p = jnp.pad(fc2_W, ((0, 0), (0, 8 - fc2_W.shape[1])))
    fb2p = jnp.pad(fc2_b, ((0, 8 - fc2_b.shape[0]),))
    out8 = _pool(s2, batch3, es8, c2_bias, fc1_W, fc1_b, fc2p, fb2p,
                 bn, hc, ng)
    return out8[:, :fc2_W.shape[1]]


def _build_atts(att, h_, c_):
    a = jnp.zeros((8, h_ * c_), F32)
    for h in range(h_):
        a = a.at[h, h * c_:(h + 1) * c_].set(att[h])
    return a.T
